# baseline (speedup 1.0000x reference)
import functools

import jax
import jax.numpy as jnp
from jax import lax
from jax.experimental import pallas as pl
from jax.experimental.pallas import tpu as pltpu
from jax.experimental.pallas import tpu_sc as plsc

B, T, V = 4, 2048, 8192
N = B * T
NC, NS, L = 2, 16, 16
NW = NC * NS
PER_W = N // NW
CHUNK = 128
NCHUNK = PER_W // CHUNK

_mesh = plsc.VectorSubcoreMesh(core_axis_name="c", subcore_axis_name="s")


@functools.partial(
    pl.kernel,
    mesh=_mesh,
    out_type=jax.ShapeDtypeStruct((NW, L), jnp.float32),
    scratch_types=[
        pltpu.VMEM((PER_W,), jnp.int32),
        pltpu.VMEM((NCHUNK, CHUNK), jnp.int32),
        pltpu.VMEM((NCHUNK, CHUNK), jnp.float32),
        pltpu.VMEM((L,), jnp.float32),
        pltpu.SemaphoreType.DMA,
    ],
)
def _gather_neg_sum(tgt_hbm, flat_hbm, out_hbm, tgt_v, idx_v, val_v, acc_v, sem):
    wid = lax.axis_index("s") * NC + lax.axis_index("c")
    base = wid * PER_W
    pltpu.sync_copy(tgt_hbm.at[pl.ds(base, PER_W)], tgt_v)

    lanes = lax.iota(jnp.int32, L)
    for c in range(NCHUNK):
        for j in range(CHUNK // L):
            off = c * CHUNK + j * L
            idx_v[jnp.int32(c), pl.ds(j * L, L)] = (
                (base + off + lanes) * V + tgt_v[pl.ds(off, L)]
            )

    copies = [
        pltpu.async_copy(
            flat_hbm.at[idx_v.at[jnp.int32(c)]], val_v.at[jnp.int32(c)], sem
        )
        for c in range(NCHUNK)
    ]
    for cp in copies:
        cp.wait()

    acc = jnp.zeros((L,), jnp.float32)
    for c in range(NCHUNK):
        for j in range(CHUNK // L):
            acc = acc - val_v[jnp.int32(c), pl.ds(j * L, L)]
    acc_v[...] = acc
    pltpu.sync_copy(acc_v, out_hbm.at[wid])


def kernel(model, sample, precomputed, target):
    del model, sample
    tgt = target.reshape(N).astype(jnp.int32)
    flat = precomputed.reshape(N * V)
    partials = _gather_neg_sum(tgt, flat)
    return jnp.sum(partials)


# speedup vs baseline: 5.0558x; 5.0558x over previous
import functools

import jax
import jax.numpy as jnp
from jax import lax
from jax.experimental import pallas as pl
from jax.experimental.pallas import tpu as pltpu
from jax.experimental.pallas import tpu_sc as plsc

B, T, V = 4, 2048, 8192
N = B * T
NC, NS, L = 2, 16, 16
NW = NC * NS
PER_W = N // NW
WAVE = 64

_mesh = plsc.VectorSubcoreMesh(core_axis_name="c", subcore_axis_name="s")


@functools.partial(
    pl.kernel,
    mesh=_mesh,
    out_type=jax.ShapeDtypeStruct((NW, L), jnp.float32),
    scratch_types=[
        pltpu.VMEM((PER_W,), jnp.int32),
        pltpu.VMEM((WAVE, 8, 128), jnp.float32),
        pltpu.VMEM((L,), jnp.float32),
        pltpu.SemaphoreType.DMA,
    ],
    compiler_params=pltpu.CompilerParams(
        use_tc_tiling_on_sc=True, needs_layout_passes=False
    ),
)
def _gather_neg_sum(tgt_hbm, y_hbm, out_hbm, tgt_v, val_v, acc_v, sem):
    wid = lax.axis_index("s") * NC + lax.axis_index("c")
    base = wid * PER_W
    pltpu.sync_copy(tgt_hbm.at[pl.ds(base, PER_W)], tgt_v)

    lanes = lax.iota(jnp.int32, L)
    acc = jnp.zeros((L,), jnp.float32)
    for g in range(PER_W // WAVE):
        chunks = [tgt_v[pl.ds(g * WAVE + k * L, L)] for k in range(WAVE // L)]
        copies = []
        for j in range(WAVE):
            i = g * WAVE + j
            t = chunks[j // L][j % L]
            c0 = pl.multiple_of((t >> 7) << 7, 128)
            r0 = base + (i // 8) * 8
            copies.append(
                pltpu.async_copy(
                    y_hbm.at[pl.ds(r0, 8), pl.ds(c0, 128)],
                    val_v.at[jnp.int32(j)],
                    sem,
                )
            )
        for cp in copies:
            cp.wait()
        for k in range(WAVE // L):
            p = jnp.int32(k * L) + lanes
            s = p & 7
            col = chunks[k] & 127
            acc = acc - plsc.load_gather(val_v, [p, s, col])
    acc_v[...] = acc
    pltpu.sync_copy(acc_v, out_hbm.at[wid])


def kernel(model, sample, precomputed, target):
    del model, sample
    tgt = target.reshape(N).astype(jnp.int32)
    y = precomputed.reshape(N, V)
    partials = _gather_neg_sum(tgt, y)
    return jnp.sum(partials)


# speedup vs baseline: 6.3523x; 1.2564x over previous
import functools

import jax
import jax.numpy as jnp
from jax import lax
from jax.experimental import pallas as pl
from jax.experimental.pallas import tpu as pltpu
from jax.experimental.pallas import tpu_sc as plsc

B, T, V = 4, 2048, 8192
N = B * T
NC, NS, L = 2, 16, 16
NW = NC * NS
PER_W = N // NW

_mesh = plsc.VectorSubcoreMesh(core_axis_name="c", subcore_axis_name="s")


@functools.partial(
    pl.kernel,
    mesh=_mesh,
    out_type=jax.ShapeDtypeStruct((NW, L), jnp.float32),
    scratch_types=[
        pltpu.VMEM((PER_W,), jnp.int32),
        pltpu.VMEM((8 * PER_W,), jnp.int32),
        pltpu.VMEM((PER_W, 128), jnp.float32),
        pltpu.VMEM((L,), jnp.float32),
        pltpu.SemaphoreType.DMA,
    ],
    compiler_params=pltpu.CompilerParams(
        use_tc_tiling_on_sc=True, needs_layout_passes=False
    ),
)
def _gather_neg_sum(tgt_hbm, y_hbm, out_hbm, tgt_v, row_v, val_v, acc_v, sem):
    wid = lax.axis_index("s") * NC + lax.axis_index("c")
    base = wid * PER_W

    pltpu.sync_copy(tgt_hbm.at[pl.ds(base, PER_W)], tgt_v)
    lanes = lax.iota(jnp.int32, L)
    for j in range(PER_W // L):
        plsc.store_scatter(row_v, [(j * L + lanes) * 8], base + j * L + lanes)

    chunks = [tgt_v[pl.ds(k * L, L)] for k in range(PER_W // L)]
    copies = []
    for i in range(PER_W):
        t = chunks[i // L][i % L]
        c0 = pl.multiple_of((t >> 7) << 7, 128)
        copies.append(
            pltpu.async_copy(
                y_hbm.at[row_v.at[pl.ds(i * 8, 1)], pl.ds(c0, 128)],
                val_v.at[pl.ds(i, 1)],
                sem,
            )
        )
    for cp in copies:
        cp.wait()

    acc = jnp.zeros((L,), jnp.float32)
    for j in range(PER_W // L):
        p = jnp.int32(j * L) + lanes
        acc = acc - plsc.load_gather(val_v, [p, chunks[j] & 127])
    acc_v[...] = acc
    pltpu.sync_copy(acc_v, out_hbm.at[wid])


def kernel(model, sample, precomputed, target):
    del model, sample
    tgt = target.reshape(N).astype(jnp.int32)
    y = precomputed.reshape(N, V)
    partials = _gather_neg_sum(tgt, y)
    return jnp.sum(partials)


# speedup vs baseline: 6.8531x; 1.0788x over previous
import functools

import jax
import jax.numpy as jnp
from jax import lax
from jax.experimental import pallas as pl
from jax.experimental.pallas import tpu as pltpu
from jax.experimental.pallas import tpu_sc as plsc

B, T, V = 4, 2048, 8192
N = B * T
NC, NS, L = 2, 16, 16
NW = NC * NS
PER_W = N // NW

_mesh = plsc.VectorSubcoreMesh(core_axis_name="c", subcore_axis_name="s")


@functools.partial(
    pl.kernel,
    mesh=_mesh,
    out_type=jax.ShapeDtypeStruct((NW, L), jnp.float32),
    scratch_types=[
        pltpu.VMEM((PER_W,), jnp.int32),
        pltpu.VMEM((8 * PER_W,), jnp.int32),
        pltpu.VMEM((PER_W, 128), jnp.float32),
        pltpu.VMEM((L,), jnp.float32),
        pltpu.SemaphoreType.DMA,
    ],
    compiler_params=pltpu.CompilerParams(
        use_tc_tiling_on_sc=True, needs_layout_passes=False
    ),
)
def _gather_neg_sum(tgt_hbm, y_hbm, out_hbm, tgt_v, row_v, val_v, acc_v, sem):
    wid = lax.axis_index("s") * NC + lax.axis_index("c")
    base = wid * PER_W

    pltpu.sync_copy(tgt_hbm.at[pl.ds(base, PER_W)], tgt_v)
    lanes = lax.iota(jnp.int32, L)
    chunks = []
    c0chunks = []
    for j in range(PER_W // L):
        t = tgt_v[pl.ds(j * L, L)]
        chunks.append(t)
        c0chunks.append(t & jnp.int32(-128))
        plsc.store_scatter(row_v, [(j * L + lanes) * 8], base + j * L + lanes)

    for i in range(PER_W):
        c0 = pl.multiple_of(c0chunks[i // L][i % L], 128)
        pltpu.async_copy(
            y_hbm.at[row_v.at[pl.ds(i * 8, 1)], pl.ds(c0, 128)],
            val_v.at[pl.ds(i, 1)],
            sem,
        )
    pltpu.make_async_copy(
        y_hbm.at[pl.ds(0, PER_W), pl.ds(0, 128)], val_v, sem
    ).wait()

    acc = jnp.zeros((L,), jnp.float32)
    for j in range(PER_W // L):
        p = jnp.int32(j * L) + lanes
        acc = acc - plsc.load_gather(val_v, [p, chunks[j] & 127])
    acc_v[...] = acc
    pltpu.sync_copy(acc_v, out_hbm.at[wid])


def kernel(model, sample, precomputed, target):
    del model, sample
    tgt = target.reshape(N).astype(jnp.int32)
    y = precomputed.reshape(N, V)
    partials = _gather_neg_sum(tgt, y)
    return jnp.sum(partials)


# speedup vs baseline: 8.4425x; 1.2319x over previous
import functools

import jax
import jax.numpy as jnp
from jax import lax
from jax.experimental import pallas as pl
from jax.experimental.pallas import tpu as pltpu
from jax.experimental.pallas import tpu_sc as plsc

B, T, V = 4, 2048, 8192
N = B * T
NC, NS, L = 2, 16, 16
NW = NC * NS
PER_W = N // NW

_mesh = plsc.VectorSubcoreMesh(core_axis_name="c", subcore_axis_name="s")


@functools.partial(
    pl.kernel,
    mesh=_mesh,
    out_type=jax.ShapeDtypeStruct((NW, L), jnp.float32),
    scratch_types=[
        pltpu.VMEM((PER_W,), jnp.int32),
        pltpu.VMEM((8 * PER_W,), jnp.int32),
        pltpu.VMEM((PER_W, 128), jnp.float32),
        pltpu.VMEM((L,), jnp.float32),
        pltpu.VMEM((PER_W,), jnp.int32),
        pltpu.SemaphoreType.DMA,
    ],
    compiler_params=pltpu.CompilerParams(
        use_tc_tiling_on_sc=True, needs_layout_passes=False
    ),
)
def _gather_neg_sum(tgt_hbm, y_hbm, out_hbm, tgt_v, row_v, val_v, acc_v, col_v, sem):
    wid = lax.axis_index("s") * NC + lax.axis_index("c")
    base = wid * PER_W

    pltpu.sync_copy(tgt_hbm.at[pl.ds(base, PER_W)], tgt_v)
    lanes = lax.iota(jnp.int32, L)
    chunks = []
    for j in range(PER_W // L):
        t = tgt_v[pl.ds(j * L, L)]
        chunks.append(t)
        plsc.store_scatter(row_v, [(j * L + lanes) * 8], base + j * L + lanes)
        col_v[pl.ds(j * L, L)] = t & jnp.int32(-128)

    UNROLL = 8

    def _issue(g, carry):
        i0 = g * UNROLL
        c0vec = plsc.load_gather(col_v, [(lanes & 7) + i0])
        for u in range(UNROLL):
            i = i0 + jnp.int32(u)
            c0 = pl.multiple_of(c0vec[u], 128)
            pltpu.async_copy(
                y_hbm.at[
                    row_v.at[pl.ds(pl.multiple_of(i * 8, 8), 1)],
                    pl.ds(c0, 128),
                ],
                val_v.at[pl.ds(i, 1)],
                sem,
            )
        return carry

    lax.fori_loop(jnp.int32(0), jnp.int32(PER_W // UNROLL), _issue, jnp.int32(0))
    pltpu.make_async_copy(
        y_hbm.at[pl.ds(0, PER_W), pl.ds(0, 128)], val_v, sem
    ).wait()

    acc = jnp.zeros((L,), jnp.float32)
    for j in range(PER_W // L):
        p = jnp.int32(j * L) + lanes
        acc = acc - plsc.load_gather(val_v, [p, chunks[j] & 127])
    acc_v[...] = acc
    pltpu.sync_copy(acc_v, out_hbm.at[wid])


def kernel(model, sample, precomputed, target):
    del model, sample
    tgt = target.reshape(N).astype(jnp.int32)
    y = precomputed.reshape(N, V)
    partials = _gather_neg_sum(tgt, y)
    return jnp.sum(partials)
